# baseline (speedup 1.0000x reference)
import jax
from jax.experimental import pallas as pl

_GRID = 10


def _copy_body(x_ref, ea_ref, u_ref, xo_ref, eao_ref, uo_ref):
    xo_ref[...] = x_ref[...]
    eao_ref[...] = ea_ref[...]
    uo_ref[...] = u_ref[...]


def kernel(x, edge_index, edge_attr, u, batch, queries, num_props):
    ea_shape = edge_attr.shape
    ea = edge_attr.reshape(-1, 128)
    n_x = x.shape[0] // _GRID
    n_ea = ea.shape[0] // _GRID
    xs = pl.BlockSpec((n_x, x.shape[1]), lambda i: (i, 0))
    eas = pl.BlockSpec((n_ea, ea.shape[1]), lambda i: (i, 0))
    us = pl.BlockSpec(u.shape, lambda i: (0, 0))
    outs = pl.pallas_call(
        _copy_body,
        grid=(_GRID,),
        out_shape=(
            jax.ShapeDtypeStruct(x.shape, x.dtype),
            jax.ShapeDtypeStruct(ea.shape, ea.dtype),
            jax.ShapeDtypeStruct(u.shape, u.dtype),
        ),
        in_specs=[xs, eas, us],
        out_specs=(xs, eas, us),
    )(x, ea, u)
    return (outs[0], outs[1].reshape(ea_shape), outs[2])


# speedup vs baseline: 1.0085x; 1.0085x over previous
import functools

import jax
from jax import lax
from jax.experimental import pallas as pl
from jax.experimental.pallas import tpu as pltpu
from jax.experimental.pallas import tpu_sc as plsc

_GRID = 10
_CHUNKS = 10


def _xu_body(x_ref, u_ref, xo_ref, uo_ref):
    xo_ref[...] = x_ref[...]
    uo_ref[...] = u_ref[...]


def _copy_xu(x, u):
    n_x = x.shape[0] // _GRID
    xs = pl.BlockSpec((n_x, x.shape[1]), lambda i: (i, 0))
    us = pl.BlockSpec(u.shape, lambda i: (0, 0))
    return pl.pallas_call(
        _xu_body,
        grid=(_GRID,),
        out_shape=(
            jax.ShapeDtypeStruct(x.shape, x.dtype),
            jax.ShapeDtypeStruct(u.shape, u.dtype),
        ),
        in_specs=[xs, us],
        out_specs=(xs, us),
    )(x, u)


def _copy_ea(edge_attr):
    info = plsc.get_sparse_core_info()
    n_workers = info.num_cores * info.num_subcores
    rows_w = edge_attr.shape[0] // n_workers
    rows_c = rows_w // _CHUNKS
    mesh = plsc.VectorSubcoreMesh(core_axis_name="c", subcore_axis_name="s")

    @functools.partial(
        pl.kernel,
        mesh=mesh,
        out_type=jax.ShapeDtypeStruct(edge_attr.shape, edge_attr.dtype),
        scratch_types=[
            pltpu.VMEM((rows_c, edge_attr.shape[1]), edge_attr.dtype),
        ],
    )
    def _ea_kernel(ea_hbm, out_hbm, buf):
        wid = lax.axis_index("s") * info.num_cores + lax.axis_index("c")
        base = wid * rows_w
        for k in range(_CHUNKS):
            start = pl.multiple_of(base + k * rows_c, 8)
            pltpu.sync_copy(ea_hbm.at[pl.ds(start, rows_c), :], buf)
            pltpu.sync_copy(buf, out_hbm.at[pl.ds(start, rows_c), :])

    return _ea_kernel(edge_attr)


def kernel(x, edge_index, edge_attr, u, batch, queries, num_props):
    xo, uo = _copy_xu(x, u)
    eao = _copy_ea(edge_attr)
    return (xo, eao, uo)
